# baseline (speedup 1.0000x reference)
import functools

import jax
import jax.numpy as jnp
from jax import lax
from jax.experimental import pallas as pl
from jax.experimental.pallas import tpu as pltpu
from jax.experimental.pallas import tpu_sc as plsc

VOCAB = 1_000_000
MAXLEN = 200
EMBED = 64
BATCH = 4096

_INFO = plsc.get_sparse_core_info()
NC, NS = _INFO.num_cores, _INFO.num_subcores
NW = NC * NS
HALF = 100
NHALF = BATCH * MAXLEN // HALF
H_PER_W = NHALF // NW
K = 4
G = H_PER_W // K
CHUNK = K * HALF


def _body(x_hbm, tok_hbm, pos_hbm, out_hbm, idx_v, pos_v, buf_v, gsem, osem):
    wid = lax.axis_index("s") * NC + lax.axis_index("c")

    pltpu.sync_copy(pos_hbm, pos_v.at[pl.ds(0, MAXLEN)])
    pltpu.sync_copy(pos_hbm, pos_v.at[pl.ds(MAXLEN, MAXLEN)])

    def fire(g, s):
        hs0 = wid * H_PER_W + g * K
        pltpu.sync_copy(x_hbm.at[pl.ds(hs0, K)], idx_v.at[s])
        for k in range(K):
            pltpu.async_copy(tok_hbm.at[idx_v.at[s, k]],
                             buf_v.at[s, pl.ds(k * HALF, HALF)], gsem.at[s])

    def drain_gathers(s):
        for k in range(K):
            pltpu.make_async_copy(tok_hbm.at[idx_v.at[s, k]],
                                  buf_v.at[s, pl.ds(k * HALF, HALF)],
                                  gsem.at[s]).wait()

    def add_pos(s):
        def add_rows(i, c2):
            for rr in range(4):
                r = i * 4 + rr
                for j in range(EMBED // 16):
                    plsc.addupdate(buf_v.at[s, r, pl.ds(j * 16, 16)],
                                   pos_v[r, pl.ds(j * 16, 16)])
            return c2
        lax.fori_loop(0, CHUNK // 4, add_rows, 0, unroll=4)

    def out_copy(g, s):
        row0 = (wid * H_PER_W + g * K) * HALF
        pltpu.async_copy(buf_v.at[s], out_hbm.at[pl.ds(row0, CHUNK)], osem.at[s])

    def wait_out(s):
        pltpu.make_async_copy(buf_v.at[s],
                              out_hbm.at[pl.ds(0, CHUNK)], osem.at[s]).wait()

    def step(g, carry):
        for s in (0, 1):
            gg = g * 2 + s
            o = 1 - s

            @pl.when(gg >= 2)
            def _():
                wait_out(s)

            fire(gg, s)

            @pl.when(gg >= 1)
            def _():
                drain_gathers(o)
                add_pos(o)
                out_copy(gg - 1, o)
        return carry

    lax.fori_loop(0, G // 2, step, 0)

    drain_gathers(1)
    add_pos(1)
    out_copy(G - 1, 1)
    wait_out(0)
    wait_out(1)


_TB = 6400


def _tc_transpose_block(in_ref, out_ref):
    out_ref[...] = in_ref[...].T


def _retile_table(token_table):
    tt2 = token_table.T
    return pl.pallas_call(
        _tc_transpose_block,
        grid=(pl.cdiv(VOCAB, _TB),),
        in_specs=[pl.BlockSpec((EMBED, _TB), lambda i: (0, i))],
        out_specs=pl.BlockSpec((_TB, EMBED), lambda i: (i, 0)),
        out_shape=jax.ShapeDtypeStruct((VOCAB, EMBED), jnp.float32),
    )(tt2)


def kernel(x, token_table, pos_table):
    x2 = x.reshape(NHALF, HALF).astype(jnp.int32)
    token_rm = _retile_table(token_table)
    mesh = plsc.VectorSubcoreMesh(core_axis_name="c", subcore_axis_name="s")
    run = functools.partial(
        pl.kernel,
        out_type=jax.ShapeDtypeStruct((BATCH * MAXLEN, EMBED), jnp.float32),
        mesh=mesh,
        compiler_params=pltpu.CompilerParams(use_tc_tiling_on_sc=False),
        scratch_types=[
            pltpu.VMEM((2, K, HALF), jnp.int32),
            pltpu.VMEM((CHUNK, EMBED), jnp.float32),
            pltpu.VMEM((2, CHUNK, EMBED), jnp.float32),
            pltpu.SemaphoreType.DMA((2,)),
            pltpu.SemaphoreType.DMA((2,)),
        ],
    )(_body)
    out = run(x2, token_rm, pos_table)
    return out.reshape(BATCH, MAXLEN, EMBED)


# speedup vs baseline: 1.4776x; 1.4776x over previous
import functools

import jax
import jax.numpy as jnp
from jax import lax
from jax.experimental import pallas as pl
from jax.experimental.pallas import tpu as pltpu
from jax.experimental.pallas import tpu_sc as plsc

VOCAB = 1_000_000
MAXLEN = 200
EMBED = 64
BATCH = 4096

_INFO = plsc.get_sparse_core_info()
NC, NS = _INFO.num_cores, _INFO.num_subcores
NW = NC * NS
HALF = 100
NHALF = BATCH * MAXLEN // HALF
H_PER_W = NHALF // NW
K = 4
G = H_PER_W // K
CHUNK = K * HALF


def _body(x_hbm, tok_hbm, pos_hbm, out_hbm, idx_v, pos_v, buf_v, gsem, osem):
    wid = lax.axis_index("s") * NC + lax.axis_index("c")

    pltpu.sync_copy(pos_hbm, pos_v.at[pl.ds(0, MAXLEN)])
    pltpu.sync_copy(pos_hbm, pos_v.at[pl.ds(MAXLEN, MAXLEN)])

    def fire(g, s):
        hs0 = wid * H_PER_W + g * K
        pltpu.sync_copy(x_hbm.at[pl.ds(hs0, K)], idx_v.at[s])
        for k in range(K):
            pltpu.async_copy(tok_hbm.at[idx_v.at[s, k]],
                             buf_v.at[s, pl.ds(k * HALF, HALF)], gsem.at[s])

    def drain_gathers(s):
        for k in range(K):
            pltpu.make_async_copy(tok_hbm.at[idx_v.at[s, k]],
                                  buf_v.at[s, pl.ds(k * HALF, HALF)],
                                  gsem.at[s]).wait()

    def add_pos(s):
        def add_rows(i, c2):
            for rr in range(4):
                r = i * 4 + rr
                for j in range(EMBED // 16):
                    plsc.addupdate(buf_v.at[s, r, pl.ds(j * 16, 16)],
                                   pos_v[r, pl.ds(j * 16, 16)])
            return c2
        lax.fori_loop(0, CHUNK // 4, add_rows, 0, unroll=4)

    def out_copy(g, s):
        row0 = (wid * H_PER_W + g * K) * HALF
        pltpu.async_copy(buf_v.at[s], out_hbm.at[pl.ds(row0, CHUNK)], osem.at[s])

    def wait_out(s):
        pltpu.make_async_copy(buf_v.at[s],
                              out_hbm.at[pl.ds(0, CHUNK)], osem.at[s]).wait()

    def step(g, carry):
        for s in (0, 1):
            gg = g * 2 + s
            o = 1 - s

            @pl.when(gg >= 2)
            def _():
                wait_out(s)

            fire(gg, s)

            @pl.when(gg >= 1)
            def _():
                drain_gathers(o)
                add_pos(o)
                out_copy(gg - 1, o)
        return carry

    lax.fori_loop(0, G // 2, step, 0)

    drain_gathers(1)
    add_pos(1)
    out_copy(G - 1, 1)
    wait_out(0)
    wait_out(1)


_TB = 6400


def _tc_transpose_block(in_ref, out_ref):
    out_ref[:, 0:EMBED] = in_ref[...].T


def _retile_table(token_table):
    tt2 = token_table.T
    padded = pl.pallas_call(
        _tc_transpose_block,
        grid=(pl.cdiv(VOCAB, _TB),),
        in_specs=[pl.BlockSpec((EMBED, _TB), lambda i: (0, i))],
        out_specs=pl.BlockSpec((_TB, 2 * EMBED), lambda i: (i, 0)),
        out_shape=jax.ShapeDtypeStruct((VOCAB, 2 * EMBED), jnp.float32),
    )(tt2)
    return padded.reshape(2 * VOCAB, EMBED)


_PB = 8
_BB = 1024


def _tc_out_block(in_ref, out_ref):
    for q in range(_PB):
        out_ref[q] = in_ref[:, q * EMBED:(q + 1) * EMBED].T


def _retile_out(sc_out):
    view2 = sc_out.reshape(BATCH, MAXLEN * EMBED)
    out3 = pl.pallas_call(
        _tc_out_block,
        grid=(MAXLEN // _PB, BATCH // _BB),
        in_specs=[pl.BlockSpec((_BB, _PB * EMBED), lambda j, c: (c, j))],
        out_specs=pl.BlockSpec((_PB, EMBED, _BB), lambda j, c: (j, 0, c)),
        out_shape=jax.ShapeDtypeStruct((MAXLEN, EMBED, BATCH), jnp.float32),
    )(view2)
    return out3.transpose(2, 0, 1)


def kernel(x, token_table, pos_table):
    x2 = x.reshape(NHALF, HALF).astype(jnp.int32) * 2
    token_rm = _retile_table(token_table)
    mesh = plsc.VectorSubcoreMesh(core_axis_name="c", subcore_axis_name="s")
    run = functools.partial(
        pl.kernel,
        out_type=jax.ShapeDtypeStruct((BATCH * MAXLEN, EMBED), jnp.float32),
        mesh=mesh,
        compiler_params=pltpu.CompilerParams(use_tc_tiling_on_sc=False),
        scratch_types=[
            pltpu.VMEM((2, K, HALF), jnp.int32),
            pltpu.VMEM((CHUNK, EMBED), jnp.float32),
            pltpu.VMEM((2, CHUNK, EMBED), jnp.float32),
            pltpu.SemaphoreType.DMA((2,)),
            pltpu.SemaphoreType.DMA((2,)),
        ],
    )(_body)
    out = run(x2, token_rm, pos_table)
    return _retile_out(out)


# speedup vs baseline: 1.5356x; 1.0392x over previous
import functools

import jax
import jax.numpy as jnp
from jax import lax
from jax.experimental import pallas as pl
from jax.experimental.pallas import tpu as pltpu
from jax.experimental.pallas import tpu_sc as plsc

VOCAB = 1_000_000
MAXLEN = 200
EMBED = 64
BATCH = 4096

_INFO = plsc.get_sparse_core_info()
NC, NS = _INFO.num_cores, _INFO.num_subcores
NW = NC * NS
HALF = 100
NHALF = BATCH * MAXLEN // HALF
H_PER_W = NHALF // NW
K = 4
G = H_PER_W // K
CHUNK = K * HALF


def _body(x_hbm, tok_hbm, pos_hbm, out_hbm, idx_v, pos_v, buf_v, gsem, osem):
    wid = lax.axis_index("s") * NC + lax.axis_index("c")

    pltpu.sync_copy(pos_hbm, pos_v.at[pl.ds(0, MAXLEN)])
    pltpu.sync_copy(pos_hbm, pos_v.at[pl.ds(MAXLEN, MAXLEN)])

    def fire(g, s):
        hs0 = wid * H_PER_W + g * K
        pltpu.sync_copy(x_hbm.at[pl.ds(hs0, K)], idx_v.at[s])
        for k in range(K):
            pltpu.async_copy(tok_hbm.at[idx_v.at[s, k]],
                             buf_v.at[s, pl.ds(k * HALF, HALF)], gsem.at[s])

    def drain_gathers(s):
        for k in range(K):
            pltpu.make_async_copy(tok_hbm.at[idx_v.at[s, k]],
                                  buf_v.at[s, pl.ds(k * HALF, HALF)],
                                  gsem.at[s]).wait()

    def add_pos(s):
        def add_rows(i, c2):
            for rr in range(4):
                r = i * 4 + rr
                for j in range(EMBED // 16):
                    plsc.addupdate(buf_v.at[s, r, pl.ds(j * 16, 16)],
                                   pos_v[r, pl.ds(j * 16, 16)])
            return c2
        lax.fori_loop(0, CHUNK // 4, add_rows, 0, unroll=4)

    def out_copy(g, s):
        row0 = (wid * H_PER_W + g * K) * HALF
        pltpu.async_copy(buf_v.at[s], out_hbm.at[pl.ds(row0, CHUNK)], osem.at[s])

    def wait_out(s):
        pltpu.make_async_copy(buf_v.at[s],
                              out_hbm.at[pl.ds(0, CHUNK)], osem.at[s]).wait()

    def step(g, carry):
        for s in (0, 1):
            gg = g * 2 + s
            o = 1 - s

            @pl.when(gg >= 2)
            def _():
                wait_out(s)

            fire(gg, s)

            @pl.when(gg >= 1)
            def _():
                drain_gathers(o)
                add_pos(o)
                out_copy(gg - 1, o)
        return carry

    lax.fori_loop(0, G // 2, step, 0)

    drain_gathers(1)
    add_pos(1)
    out_copy(G - 1, 1)
    wait_out(0)
    wait_out(1)


_TB = 12800


def _tc_transpose_block(in_ref, out_ref):
    out_ref[:, 0:EMBED] = in_ref[...].T


def _retile_table(token_table):
    tt2 = token_table.T
    padded = pl.pallas_call(
        _tc_transpose_block,
        grid=(pl.cdiv(VOCAB, _TB),),
        in_specs=[pl.BlockSpec((EMBED, _TB), lambda i: (0, i))],
        out_specs=pl.BlockSpec((_TB, 2 * EMBED), lambda i: (i, 0)),
        out_shape=jax.ShapeDtypeStruct((VOCAB, 2 * EMBED), jnp.float32),
    )(tt2)
    return padded.reshape(2 * VOCAB, EMBED)


_PB = 8
_BB = 1024


def _tc_out_block(in_ref, out_ref):
    for q in range(_PB):
        out_ref[q] = in_ref[:, q * EMBED:(q + 1) * EMBED].T


def _retile_out(sc_out):
    view2 = sc_out.reshape(BATCH, MAXLEN * EMBED)
    out3 = pl.pallas_call(
        _tc_out_block,
        grid=(MAXLEN // _PB, BATCH // _BB),
        in_specs=[pl.BlockSpec((_BB, _PB * EMBED), lambda j, c: (c, j))],
        out_specs=pl.BlockSpec((_PB, EMBED, _BB), lambda j, c: (j, 0, c)),
        out_shape=jax.ShapeDtypeStruct((MAXLEN, EMBED, BATCH), jnp.float32),
    )(view2)
    return out3.transpose(2, 0, 1)


def kernel(x, token_table, pos_table):
    x2 = x.reshape(NHALF, HALF).astype(jnp.int32) * 2
    token_rm = _retile_table(token_table)
    mesh = plsc.VectorSubcoreMesh(core_axis_name="c", subcore_axis_name="s")
    run = functools.partial(
        pl.kernel,
        out_type=jax.ShapeDtypeStruct((BATCH * MAXLEN, EMBED), jnp.float32),
        mesh=mesh,
        compiler_params=pltpu.CompilerParams(use_tc_tiling_on_sc=False),
        scratch_types=[
            pltpu.VMEM((2, K, HALF), jnp.int32),
            pltpu.VMEM((CHUNK, EMBED), jnp.float32),
            pltpu.VMEM((2, CHUNK, EMBED), jnp.float32),
            pltpu.SemaphoreType.DMA((2,)),
            pltpu.SemaphoreType.DMA((2,)),
        ],
    )(_body)
    out = run(x2, token_rm, pos_table)
    return _retile_out(out)


# speedup vs baseline: 1.5377x; 1.0014x over previous
import functools

import jax
import jax.numpy as jnp
from jax import lax
from jax.experimental import pallas as pl
from jax.experimental.pallas import tpu as pltpu
from jax.experimental.pallas import tpu_sc as plsc

VOCAB = 1_000_000
MAXLEN = 200
EMBED = 64
BATCH = 4096

_INFO = plsc.get_sparse_core_info()
NC, NS = _INFO.num_cores, _INFO.num_subcores
NW = NC * NS
HALF = 100
NHALF = BATCH * MAXLEN // HALF
H_PER_W = NHALF // NW
K = 4
G = H_PER_W // K
CHUNK = K * HALF


def _body(x_hbm, tok_hbm, pos_hbm, out_hbm, idx_v, pos_v, buf_v, gsem, osem):
    wid = lax.axis_index("s") * NC + lax.axis_index("c")

    pltpu.sync_copy(pos_hbm, pos_v.at[pl.ds(0, MAXLEN)])
    pltpu.sync_copy(pos_hbm, pos_v.at[pl.ds(MAXLEN, MAXLEN)])

    def fire(g, s):
        hs0 = wid * H_PER_W + g * K
        pltpu.sync_copy(x_hbm.at[pl.ds(hs0, K)], idx_v.at[s])
        for k in range(K):
            pltpu.async_copy(tok_hbm.at[idx_v.at[s, k]],
                             buf_v.at[s, pl.ds(k * HALF, HALF)], gsem.at[s])

    def drain_gathers(s):
        for k in range(K):
            pltpu.make_async_copy(tok_hbm.at[idx_v.at[s, k]],
                                  buf_v.at[s, pl.ds(k * HALF, HALF)],
                                  gsem.at[s]).wait()

    def add_pos(s):
        def add_rows(i, c2):
            for rr in range(4):
                r = i * 4 + rr
                for j in range(EMBED // 16):
                    plsc.addupdate(buf_v.at[s, r, pl.ds(j * 16, 16)],
                                   pos_v[r, pl.ds(j * 16, 16)])
            return c2
        lax.fori_loop(0, CHUNK // 4, add_rows, 0, unroll=4)

    def out_copy(g, s):
        row0 = (wid * H_PER_W + g * K) * HALF
        pltpu.async_copy(buf_v.at[s], out_hbm.at[pl.ds(row0, CHUNK)], osem.at[s])

    def wait_out(s):
        pltpu.make_async_copy(buf_v.at[s],
                              out_hbm.at[pl.ds(0, CHUNK)], osem.at[s]).wait()

    def step(g, carry):
        for s in (0, 1):
            gg = g * 2 + s
            o = 1 - s

            @pl.when(gg >= 2)
            def _():
                wait_out(s)

            fire(gg, s)

            @pl.when(gg >= 1)
            def _():
                drain_gathers(o)
                add_pos(o)
                out_copy(gg - 1, o)
        return carry

    lax.fori_loop(0, G // 2, step, 0)

    drain_gathers(1)
    add_pos(1)
    out_copy(G - 1, 1)
    wait_out(0)
    wait_out(1)


_TB = 12800


def _tc_transpose_block(in_ref, out_ref):
    out_ref[:, 0:EMBED] = in_ref[...].T


def _retile_table(token_table):
    tt2 = token_table.T
    padded = pl.pallas_call(
        _tc_transpose_block,
        grid=(pl.cdiv(VOCAB, _TB),),
        in_specs=[pl.BlockSpec((EMBED, _TB), lambda i: (0, i))],
        out_specs=pl.BlockSpec((_TB, 2 * EMBED), lambda i: (i, 0)),
        out_shape=jax.ShapeDtypeStruct((VOCAB, 2 * EMBED), jnp.float32),
    )(tt2)
    return padded.reshape(2 * VOCAB, EMBED)


_PB = 2
_BB = 4096


def _tc_out_block(in_ref, out_ref):
    for q in range(_PB):
        out_ref[q] = in_ref[:, q * EMBED:(q + 1) * EMBED].T


def _retile_out(sc_out):
    view2 = sc_out.reshape(BATCH, MAXLEN * EMBED)
    out3 = pl.pallas_call(
        _tc_out_block,
        grid=(MAXLEN // _PB, BATCH // _BB),
        in_specs=[pl.BlockSpec((_BB, _PB * EMBED), lambda j, c: (c, j))],
        out_specs=pl.BlockSpec((_PB, EMBED, _BB), lambda j, c: (j, 0, c)),
        out_shape=jax.ShapeDtypeStruct((MAXLEN, EMBED, BATCH), jnp.float32),
    )(view2)
    return out3.transpose(2, 0, 1)


def kernel(x, token_table, pos_table):
    x2 = x.reshape(NHALF, HALF).astype(jnp.int32) * 2
    token_rm = _retile_table(token_table)
    mesh = plsc.VectorSubcoreMesh(core_axis_name="c", subcore_axis_name="s")
    run = functools.partial(
        pl.kernel,
        out_type=jax.ShapeDtypeStruct((BATCH * MAXLEN, EMBED), jnp.float32),
        mesh=mesh,
        compiler_params=pltpu.CompilerParams(use_tc_tiling_on_sc=False),
        scratch_types=[
            pltpu.VMEM((2, K, HALF), jnp.int32),
            pltpu.VMEM((CHUNK, EMBED), jnp.float32),
            pltpu.VMEM((2, CHUNK, EMBED), jnp.float32),
            pltpu.SemaphoreType.DMA((2,)),
            pltpu.SemaphoreType.DMA((2,)),
        ],
    )(_body)
    out = run(x2, token_rm, pos_table)
    return _retile_out(out)


# speedup vs baseline: 1.5515x; 1.0090x over previous
import functools

import jax
import jax.numpy as jnp
from jax import lax
from jax.experimental import pallas as pl
from jax.experimental.pallas import tpu as pltpu
from jax.experimental.pallas import tpu_sc as plsc

VOCAB = 1_000_000
MAXLEN = 200
EMBED = 64
BATCH = 4096

_INFO = plsc.get_sparse_core_info()
NC, NS = _INFO.num_cores, _INFO.num_subcores
NW = NC * NS
HALF = 100
NHALF = BATCH * MAXLEN // HALF
H_PER_W = NHALF // NW
K = 4
G = H_PER_W // K
CHUNK = K * HALF


def _body(x_hbm, tok_hbm, pos_hbm, out_hbm, idx_v, pos_v, buf_v, gsem, osem):
    wid = lax.axis_index("s") * NC + lax.axis_index("c")

    pltpu.sync_copy(pos_hbm, pos_v.at[pl.ds(0, MAXLEN)])
    pltpu.sync_copy(pos_hbm, pos_v.at[pl.ds(MAXLEN, MAXLEN)])

    def fire(g, s):
        hs0 = wid * H_PER_W + g * K
        pltpu.sync_copy(x_hbm.at[pl.ds(hs0, K)], idx_v.at[s])
        for k in range(K):
            pltpu.async_copy(tok_hbm.at[idx_v.at[s, k]],
                             buf_v.at[s, pl.ds(k * HALF, HALF)], gsem.at[s])

    def drain_gathers(s):
        for k in range(K):
            pltpu.make_async_copy(tok_hbm.at[idx_v.at[s, k]],
                                  buf_v.at[s, pl.ds(k * HALF, HALF)],
                                  gsem.at[s]).wait()

    def add_pos(s):
        def add_rows(i, c2):
            for rr in range(4):
                r = i * 4 + rr
                for j in range(EMBED // 16):
                    plsc.addupdate(buf_v.at[s, r, pl.ds(j * 16, 16)],
                                   pos_v[r, pl.ds(j * 16, 16)])
            return c2
        lax.fori_loop(0, CHUNK // 4, add_rows, 0, unroll=4)

    def out_copy(g, s):
        row0 = (wid * H_PER_W + g * K) * HALF
        pltpu.async_copy(buf_v.at[s], out_hbm.at[pl.ds(row0, CHUNK)], osem.at[s])

    def wait_out(s):
        pltpu.make_async_copy(buf_v.at[s],
                              out_hbm.at[pl.ds(0, CHUNK)], osem.at[s]).wait()

    def step(g, carry):
        for s in (0, 1):
            gg = g * 2 + s
            o = 1 - s

            @pl.when(gg >= 2)
            def _():
                wait_out(s)

            fire(gg, s)

            @pl.when(gg >= 1)
            def _():
                drain_gathers(o)
                add_pos(o)
                out_copy(gg - 1, o)
        return carry

    lax.fori_loop(0, G // 2, step, 0)

    drain_gathers(1)
    add_pos(1)
    out_copy(G - 1, 1)
    wait_out(0)
    wait_out(1)


_TB = 25600


def _tc_transpose_block(in_ref, out_ref):
    out_ref[:, 0:EMBED] = in_ref[...].T


def _retile_table(token_table):
    tt2 = token_table.T
    padded = pl.pallas_call(
        _tc_transpose_block,
        grid=(pl.cdiv(VOCAB, _TB),),
        in_specs=[pl.BlockSpec((EMBED, _TB), lambda i: (0, i))],
        out_specs=pl.BlockSpec((_TB, 2 * EMBED), lambda i: (i, 0)),
        out_shape=jax.ShapeDtypeStruct((VOCAB, 2 * EMBED), jnp.float32),
    )(tt2)
    return padded.reshape(2 * VOCAB, EMBED)


_PB = 2
_BB = 4096


def _tc_out_block(in_ref, out_ref):
    for q in range(_PB):
        out_ref[q] = in_ref[:, q * EMBED:(q + 1) * EMBED].T


def _retile_out(sc_out):
    view2 = sc_out.reshape(BATCH, MAXLEN * EMBED)
    out3 = pl.pallas_call(
        _tc_out_block,
        grid=(MAXLEN // _PB, BATCH // _BB),
        in_specs=[pl.BlockSpec((_BB, _PB * EMBED), lambda j, c: (c, j))],
        out_specs=pl.BlockSpec((_PB, EMBED, _BB), lambda j, c: (j, 0, c)),
        out_shape=jax.ShapeDtypeStruct((MAXLEN, EMBED, BATCH), jnp.float32),
    )(view2)
    return out3.transpose(2, 0, 1)


def kernel(x, token_table, pos_table):
    x2 = x.reshape(NHALF, HALF).astype(jnp.int32) * 2
    token_rm = _retile_table(token_table)
    mesh = plsc.VectorSubcoreMesh(core_axis_name="c", subcore_axis_name="s")
    run = functools.partial(
        pl.kernel,
        out_type=jax.ShapeDtypeStruct((BATCH * MAXLEN, EMBED), jnp.float32),
        mesh=mesh,
        compiler_params=pltpu.CompilerParams(use_tc_tiling_on_sc=False),
        scratch_types=[
            pltpu.VMEM((2, K, HALF), jnp.int32),
            pltpu.VMEM((CHUNK, EMBED), jnp.float32),
            pltpu.VMEM((2, CHUNK, EMBED), jnp.float32),
            pltpu.SemaphoreType.DMA((2,)),
            pltpu.SemaphoreType.DMA((2,)),
        ],
    )(_body)
    out = run(x2, token_rm, pos_table)
    return _retile_out(out)


# speedup vs baseline: 1.5535x; 1.0013x over previous
import functools

import jax
import jax.numpy as jnp
from jax import lax
from jax.experimental import pallas as pl
from jax.experimental.pallas import tpu as pltpu
from jax.experimental.pallas import tpu_sc as plsc

VOCAB = 1_000_000
MAXLEN = 200
EMBED = 64
BATCH = 4096

_INFO = plsc.get_sparse_core_info()
NC, NS = _INFO.num_cores, _INFO.num_subcores
NW = NC * NS
HALF = 100
NHALF = BATCH * MAXLEN // HALF
H_PER_W = NHALF // NW
K = 4
G = H_PER_W // K
CHUNK = K * HALF


def _body(x_hbm, tok_hbm, pos_hbm, out_hbm, idx_v, pos_v, buf_v, gsem, osem):
    wid = lax.axis_index("s") * NC + lax.axis_index("c")

    pltpu.sync_copy(pos_hbm, pos_v.at[pl.ds(0, MAXLEN)])
    pltpu.sync_copy(pos_hbm, pos_v.at[pl.ds(MAXLEN, MAXLEN)])

    def fire(g, s):
        hs0 = wid * H_PER_W + g * K
        pltpu.sync_copy(x_hbm.at[pl.ds(hs0, K)], idx_v.at[s])
        for k in range(K):
            pltpu.async_copy(tok_hbm.at[idx_v.at[s, k]],
                             buf_v.at[s, pl.ds(k * HALF, HALF)], gsem.at[s])

    def drain_gathers(s):
        for k in range(K):
            pltpu.make_async_copy(tok_hbm.at[idx_v.at[s, k]],
                                  buf_v.at[s, pl.ds(k * HALF, HALF)],
                                  gsem.at[s]).wait()

    def add_pos(s):
        def add_rows(i, c2):
            for rr in range(4):
                r = i * 4 + rr
                for j in range(EMBED // 16):
                    plsc.addupdate(buf_v.at[s, r, pl.ds(j * 16, 16)],
                                   pos_v[r, pl.ds(j * 16, 16)])
            return c2
        lax.fori_loop(0, CHUNK // 4, add_rows, 0, unroll=4)

    def out_copy(g, s):
        row0 = (wid * H_PER_W + g * K) * HALF
        pltpu.async_copy(buf_v.at[s], out_hbm.at[pl.ds(row0, CHUNK)], osem.at[s])

    def wait_out(s):
        pltpu.make_async_copy(buf_v.at[s],
                              out_hbm.at[pl.ds(0, CHUNK)], osem.at[s]).wait()

    def step(g, carry):
        for s in (0, 1):
            gg = g * 2 + s
            o = 1 - s

            @pl.when(gg >= 2)
            def _():
                wait_out(s)

            fire(gg, s)

            @pl.when(gg >= 1)
            def _():
                drain_gathers(o)
                add_pos(o)
                out_copy(gg - 1, o)
        return carry

    lax.fori_loop(0, G // 2, step, 0)

    drain_gathers(1)
    add_pos(1)
    out_copy(G - 1, 1)
    wait_out(0)
    wait_out(1)


_TB = 32000


def _tc_transpose_block(in_ref, out_ref):
    out_ref[:, 0:EMBED] = in_ref[...].T


def _retile_table(token_table):
    tt2 = token_table.T
    padded = pl.pallas_call(
        _tc_transpose_block,
        grid=(pl.cdiv(VOCAB, _TB),),
        in_specs=[pl.BlockSpec((EMBED, _TB), lambda i: (0, i))],
        out_specs=pl.BlockSpec((_TB, 2 * EMBED), lambda i: (i, 0)),
        out_shape=jax.ShapeDtypeStruct((VOCAB, 2 * EMBED), jnp.float32),
    )(tt2)
    return padded.reshape(2 * VOCAB, EMBED)


_PB = 2
_BB = 4096


def _tc_out_block(in_ref, out_ref):
    for q in range(_PB):
        out_ref[q] = in_ref[:, q * EMBED:(q + 1) * EMBED].T


def _retile_out(sc_out):
    view2 = sc_out.reshape(BATCH, MAXLEN * EMBED)
    out3 = pl.pallas_call(
        _tc_out_block,
        grid=(MAXLEN // _PB, BATCH // _BB),
        in_specs=[pl.BlockSpec((_BB, _PB * EMBED), lambda j, c: (c, j))],
        out_specs=pl.BlockSpec((_PB, EMBED, _BB), lambda j, c: (j, 0, c)),
        out_shape=jax.ShapeDtypeStruct((MAXLEN, EMBED, BATCH), jnp.float32),
    )(view2)
    return out3.transpose(2, 0, 1)


def kernel(x, token_table, pos_table):
    x2 = x.reshape(NHALF, HALF).astype(jnp.int32) * 2
    token_rm = _retile_table(token_table)
    mesh = plsc.VectorSubcoreMesh(core_axis_name="c", subcore_axis_name="s")
    run = functools.partial(
        pl.kernel,
        out_type=jax.ShapeDtypeStruct((BATCH * MAXLEN, EMBED), jnp.float32),
        mesh=mesh,
        compiler_params=pltpu.CompilerParams(use_tc_tiling_on_sc=False),
        scratch_types=[
            pltpu.VMEM((2, K, HALF), jnp.int32),
            pltpu.VMEM((CHUNK, EMBED), jnp.float32),
            pltpu.VMEM((2, CHUNK, EMBED), jnp.float32),
            pltpu.SemaphoreType.DMA((2,)),
            pltpu.SemaphoreType.DMA((2,)),
        ],
    )(_body)
    out = run(x2, token_rm, pos_table)
    return _retile_out(out)
